# baseline (speedup 1.0000x reference)
import dataclasses
import functools
import math

import jax
import jax.numpy as jnp
from jax import lax
from jax.experimental import pallas as pl
from jax.experimental.pallas import tpu as pltpu
from jax.experimental.pallas import tpu_sc as plsc

N = 10000
E = 320000
D = 128
EB = 128
CPW = 80
NC = 2
NSUB = 16
NW = NC * NSUB
NCHUNK = NW * CPW
EPAD = NCHUNK * EB
ACC_N = 10080
BLKR = 80
NBLKZ = ACC_N // BLKR
NBLKW = N // BLKR
INV_SQRT_D = 1.0 / math.sqrt(D)
NSLAB = 79
KPAD = NSLAB * 128

_mesh = plsc.VectorSubcoreMesh(core_axis_name="c", subcore_axis_name="s")

_sc_params = pltpu.CompilerParams()
if "needs_layout_passes" in pltpu.CompilerParams.__dataclass_fields__:
    _sc_params = dataclasses.replace(_sc_params, needs_layout_passes=False)


def _zero_accumulator(sub, z_hbm, acc_sh):
    @pl.loop(sub, NBLKZ, step=NSUB)
    def _(b):
        pltpu.sync_copy(z_hbm, acc_sh.at[pl.ds(b * BLKR, BLKR)])


def _writeback(core, sub, acc_sh, out_hbm):
    @pl.loop(sub, NBLKW, step=NSUB)
    def _(b):
        pltpu.sync_copy(acc_sh.at[pl.ds(b * BLKR, BLKR)],
                        out_hbm.at[core, pl.ds(b * BLKR, BLKR)])


@jax.jit
def _sc_agg(x, src, dst, zblk):
    @functools.partial(
        pl.kernel,
        mesh=_mesh,
        out_type=jax.ShapeDtypeStruct((NC, N, D), jnp.float32),
        scratch_types=[
            pltpu.VMEM((CPW, EB), jnp.int32),
            pltpu.VMEM((CPW, EB), jnp.int32),
            pltpu.VMEM((EB, D), jnp.float32),
            pltpu.VMEM_SHARED((ACC_N, D), jnp.float32),
        ],
        compiler_params=_sc_params,
    )
    def k(x_hbm, src_hbm, dst_hbm, z_hbm, out_hbm, si_all, di_all, rows_v,
          acc_sh):
        core = lax.axis_index("c")
        sub = lax.axis_index("s")
        w = core * NSUB + sub
        _zero_accumulator(sub, z_hbm, acc_sh)
        start = w * CPW
        pltpu.sync_copy(src_hbm.at[pl.ds(start, CPW)], si_all)
        pltpu.sync_copy(dst_hbm.at[pl.ds(start, CPW)], di_all)
        plsc.subcore_barrier()

        @pl.loop(0, CPW)
        def _(t):
            pltpu.sync_copy(x_hbm.at[si_all.at[t]], rows_v)
            pltpu.sync_copy(rows_v, acc_sh.at[di_all.at[t]], add=True)

        plsc.subcore_barrier()
        _writeback(core, sub, acc_sh, out_hbm)

    return k(x, src, dst, zblk)


@jax.jit
def _sc_attn(gsc, v, s2, d2, zblk):
    @functools.partial(
        pl.kernel,
        mesh=_mesh,
        out_type=jax.ShapeDtypeStruct((NC, N, D), jnp.float32),
        scratch_types=[
            pltpu.VMEM((CPW, EB), jnp.int32),
            pltpu.VMEM((CPW, EB), jnp.int32),
            pltpu.VMEM((CPW, EB), jnp.int32),
            pltpu.VMEM((EB,), jnp.float32),
            pltpu.VMEM((EB, D), jnp.float32),
            pltpu.VMEM_SHARED((ACC_N, D), jnp.float32),
        ],
        compiler_params=_sc_params,
    )
    def k(g_hbm, v_hbm, s2_hbm, d2_hbm, z_hbm, out_hbm, si_all, di_all,
          fi_all, sc_v, vr, acc_sh):
        core = lax.axis_index("c")
        sub = lax.axis_index("s")
        w = core * NSUB + sub
        _zero_accumulator(sub, z_hbm, acc_sh)
        start = w * CPW
        pltpu.sync_copy(s2_hbm.at[pl.ds(start, CPW)], si_all)
        pltpu.sync_copy(d2_hbm.at[pl.ds(start, CPW)], di_all)

        slabw = jnp.full((16,), N * D, dtype=jnp.int32)
        dmul = jnp.full((16,), D, dtype=jnp.int32)
        seven = jnp.full((16,), 7, dtype=jnp.int32)
        low = jnp.full((16,), 127, dtype=jnp.int32)

        @pl.loop(0, CPW)
        def _(i):
            for cc in range(EB // 16):
                sl = pl.ds(cc * 16, 16)
                s16 = si_all[i, sl]
                fi_all[i, sl] = (lax.shift_right_logical(s16, seven) * slabw
                                 + di_all[i, sl] * dmul + (s16 & low))

        plsc.subcore_barrier()

        @pl.loop(0, CPW)
        def _(t):
            pltpu.sync_copy(v_hbm.at[si_all.at[t]], vr)
            pltpu.sync_copy(g_hbm.at[fi_all.at[t]], sc_v)

            @pl.loop(0, EB // 16)
            def _(jc):
                s16 = sc_v[pl.ds(jc * 16, 16)]
                for j2 in range(16):
                    lane = jnp.full((16,), j2, dtype=jnp.int32)
                    scb = jnp.take_along_axis(s16, lane, axis=0,
                                              mode="promise_in_bounds")
                    j = jc * 16 + j2
                    for cc in range(D // 16):
                        sl = pl.ds(cc * 16, 16)
                        vr[j, sl] = vr[j, sl] * scb

            pltpu.sync_copy(vr, acc_sh.at[di_all.at[t]], add=True)

        plsc.subcore_barrier()
        _writeback(core, sub, acc_sh, out_hbm)

    return k(gsc, v, s2, d2, zblk)


def _tc_qkv(x, wq, wk, wv):
    BR = 1000

    def body(x_ref, wq_ref, wk_ref, wv_ref, q_ref, k_ref, v_ref):
        xb = x_ref[...]
        q_ref[...] = jnp.dot(xb, wq_ref[...],
                             preferred_element_type=jnp.float32)
        k_ref[...] = jnp.dot(xb, wk_ref[...],
                             preferred_element_type=jnp.float32)
        v_ref[...] = jnp.dot(xb, wv_ref[...],
                             preferred_element_type=jnp.float32)

    w_spec = pl.BlockSpec((D, D), lambda i: (0, 0))
    r_spec = pl.BlockSpec((BR, D), lambda i: (i, 0))
    return pl.pallas_call(
        body,
        grid=(N // BR,),
        in_specs=[r_spec, w_spec, w_spec, w_spec],
        out_specs=[r_spec, r_spec, r_spec],
        out_shape=[jax.ShapeDtypeStruct((N, D), jnp.float32)] * 3,
    )(x, wq, wk, wv)


def _tc_scores(q, kp):
    def body(q_ref, k_ref, g_ref):
        g_ref[0] = lax.dot_general(
            q_ref[...], k_ref[...], (((1,), (1,)), ((), ())),
            preferred_element_type=jnp.float32) * INV_SQRT_D

    return pl.pallas_call(
        body,
        grid=(NSLAB,),
        in_specs=[
            pl.BlockSpec((N, D), lambda b: (0, 0)),
            pl.BlockSpec((D, D), lambda b: (b, 0)),
        ],
        out_specs=pl.BlockSpec((1, N, D), lambda b: (b, 0, 0)),
        out_shape=jax.ShapeDtypeStruct((NSLAB, N, D), jnp.float32),
    )(q, kp)


def _tc_combine(x, aggp, gatp, w0, w1):
    BR = 1000

    def body(x_ref, a_ref, g_ref, w0_ref, w1_ref, o_ref):
        xb = x_ref[...]
        agg = a_ref[0] + a_ref[1]
        gat = g_ref[0] + g_ref[1]
        gcn = (jnp.dot(xb, w0_ref[...], preferred_element_type=jnp.float32)
               + jnp.dot(agg, w1_ref[...],
                         preferred_element_type=jnp.float32))
        o_ref[...] = (gcn * (1.0 / N) + xb * ((N - 1.0) / N)
                      - gat * (1.0 / float(N) ** 3))

    return pl.pallas_call(
        body,
        grid=(N // BR,),
        in_specs=[
            pl.BlockSpec((BR, D), lambda i: (i, 0)),
            pl.BlockSpec((NC, BR, D), lambda i: (0, i, 0)),
            pl.BlockSpec((NC, BR, D), lambda i: (0, i, 0)),
            pl.BlockSpec((D, D), lambda i: (0, 0)),
            pl.BlockSpec((D, D), lambda i: (0, 0)),
        ],
        out_specs=pl.BlockSpec((BR, D), lambda i: (i, 0)),
        out_shape=jax.ShapeDtypeStruct((N, D), jnp.float32),
    )(x, aggp, gatp, w0, w1)


def _pad_edges(ei):
    s = jnp.concatenate(
        [ei[0].astype(jnp.int32), jnp.zeros((EPAD - E,), jnp.int32)])
    d = jnp.concatenate(
        [ei[1].astype(jnp.int32), jnp.full((EPAD - E,), N, jnp.int32)])
    return s.reshape(NCHUNK, EB), d.reshape(NCHUNK, EB)


def kernel(input, edge_index, edge_index_2, W0, W1, Wq, Wk, Wv):
    x = input
    src, dst = _pad_edges(edge_index)
    s2, d2 = _pad_edges(edge_index_2)
    zblk = jnp.zeros((BLKR, D), jnp.float32)
    q, k, v = _tc_qkv(x, Wq, Wk, Wv)
    kp = jnp.pad(k, ((0, KPAD - N), (0, 0)))
    gsc = _tc_scores(q, kp).reshape(NSLAB * N * D)
    aggp = _sc_agg(x, src, dst, zblk)
    zblk2 = zblk + aggp[0, :BLKR, :] * 0.0
    gatp = _sc_attn(gsc, v, s2, d2, zblk2)
    return _tc_combine(x, aggp, gatp, W0, W1)


# speedup vs baseline: 1.8646x; 1.8646x over previous
import dataclasses
import functools
import math

import jax
import jax.numpy as jnp
from jax import lax
from jax.experimental import pallas as pl
from jax.experimental.pallas import tpu as pltpu
from jax.experimental.pallas import tpu_sc as plsc

N = 10000
E = 320000
D = 128
EB = 128
NCHUNK = E // EB
NC = 2
NSUB = 16
NW = NC * NSUB
BLKR = 80
NBLK = N // BLKR
INV_SQRT_D = 1.0 / math.sqrt(D)
NSLAB = 79
KPAD = NSLAB * 128

_mesh = plsc.VectorSubcoreMesh(core_axis_name="c", subcore_axis_name="s")

_sc_params = pltpu.CompilerParams()
if "needs_layout_passes" in pltpu.CompilerParams.__dataclass_fields__:
    _sc_params = dataclasses.replace(_sc_params, needs_layout_passes=False)


def _zero_accumulator(sub, z_hbm, acc_sh):
    @pl.loop(sub, NBLK, step=NSUB)
    def _(b):
        pltpu.sync_copy(z_hbm, acc_sh.at[pl.ds(b * BLKR, BLKR)])


def _writeback(core, sub, acc_sh, out_hbm):
    @pl.loop(sub, NBLK, step=NSUB)
    def _(b):
        pltpu.sync_copy(acc_sh.at[pl.ds(b * BLKR, BLKR)],
                        out_hbm.at[core, pl.ds(b * BLKR, BLKR)])


@jax.jit
def _sc_agg(x, src, dst, zblk):
    @functools.partial(
        pl.kernel,
        mesh=_mesh,
        out_type=jax.ShapeDtypeStruct((NC, N, D), jnp.float32),
        scratch_types=[
            pltpu.VMEM((EB,), jnp.int32),
            pltpu.VMEM((EB,), jnp.int32),
            pltpu.VMEM((EB, D), jnp.float32),
            pltpu.VMEM_SHARED((N, D), jnp.float32),
        ],
        compiler_params=_sc_params,
    )
    def k(x_hbm, src_hbm, dst_hbm, z_hbm, out_hbm, si_v, di_v, rows_v,
          acc_sh):
        core = lax.axis_index("c")
        sub = lax.axis_index("s")
        w = core * NSUB + sub
        _zero_accumulator(sub, z_hbm, acc_sh)
        plsc.subcore_barrier()

        @pl.loop(w, NCHUNK, step=NW)
        def _(t):
            pltpu.sync_copy(src_hbm.at[t], si_v)
            pltpu.sync_copy(dst_hbm.at[t], di_v)
            pltpu.sync_copy(x_hbm.at[si_v], rows_v)
            pltpu.sync_copy(rows_v, acc_sh.at[di_v], add=True)

        plsc.subcore_barrier()
        _writeback(core, sub, acc_sh, out_hbm)

    return k(x, src, dst, zblk)


@jax.jit
def _sc_attn(gsc, v, s2, d2, zblk):
    @functools.partial(
        pl.kernel,
        mesh=_mesh,
        out_type=jax.ShapeDtypeStruct((NC, N, D), jnp.float32),
        scratch_types=[
            pltpu.VMEM((EB,), jnp.int32),
            pltpu.VMEM((EB,), jnp.int32),
            pltpu.VMEM((EB,), jnp.int32),
            pltpu.VMEM((EB,), jnp.float32),
            pltpu.VMEM((EB, D), jnp.float32),
            pltpu.VMEM_SHARED((N, D), jnp.float32),
        ],
        compiler_params=_sc_params,
    )
    def k(g_hbm, v_hbm, s2_hbm, d2_hbm, z_hbm, out_hbm, si_v, di_v, fi_v,
          sc_v, vr, acc_sh):
        core = lax.axis_index("c")
        sub = lax.axis_index("s")
        w = core * NSUB + sub
        _zero_accumulator(sub, z_hbm, acc_sh)
        plsc.subcore_barrier()

        @pl.loop(w, NCHUNK, step=NW)
        def _(t):
            pltpu.sync_copy(s2_hbm.at[t], si_v)
            pltpu.sync_copy(d2_hbm.at[t], di_v)
            slabw = jnp.full((16,), N * D, dtype=jnp.int32)
            dmul = jnp.full((16,), D, dtype=jnp.int32)
            seven = jnp.full((16,), 7, dtype=jnp.int32)
            low = jnp.full((16,), 127, dtype=jnp.int32)
            for cc in range(EB // 16):
                sl = pl.ds(cc * 16, 16)
                s16 = si_v[sl]
                fi_v[sl] = (lax.shift_right_logical(s16, seven) * slabw
                            + di_v[sl] * dmul + (s16 & low))
            pltpu.sync_copy(v_hbm.at[si_v], vr)
            pltpu.sync_copy(g_hbm.at[fi_v], sc_v)

            @pl.loop(0, EB // 16)
            def _(jc):
                s16 = sc_v[pl.ds(jc * 16, 16)]
                for j2 in range(16):
                    lane = jnp.full((16,), j2, dtype=jnp.int32)
                    scb = jnp.take_along_axis(s16, lane, axis=0,
                                              mode="promise_in_bounds")
                    j = jc * 16 + j2
                    for cc in range(D // 16):
                        sl = pl.ds(cc * 16, 16)
                        vr[j, sl] = vr[j, sl] * scb

            pltpu.sync_copy(vr, acc_sh.at[di_v], add=True)

        plsc.subcore_barrier()
        _writeback(core, sub, acc_sh, out_hbm)

    return k(gsc, v, s2, d2, zblk)


def _tc_qkv(x, wq, wk, wv):
    BR = 1000

    def body(x_ref, wq_ref, wk_ref, wv_ref, q_ref, k_ref, v_ref):
        xb = x_ref[...]
        q_ref[...] = jnp.dot(xb, wq_ref[...],
                             preferred_element_type=jnp.float32)
        k_ref[...] = jnp.dot(xb, wk_ref[...],
                             preferred_element_type=jnp.float32)
        v_ref[...] = jnp.dot(xb, wv_ref[...],
                             preferred_element_type=jnp.float32)

    w_spec = pl.BlockSpec((D, D), lambda i: (0, 0))
    r_spec = pl.BlockSpec((BR, D), lambda i: (i, 0))
    return pl.pallas_call(
        body,
        grid=(N // BR,),
        in_specs=[r_spec, w_spec, w_spec, w_spec],
        out_specs=[r_spec, r_spec, r_spec],
        out_shape=[jax.ShapeDtypeStruct((N, D), jnp.float32)] * 3,
    )(x, wq, wk, wv)


def _tc_scores(q, kp):
    def body(q_ref, k_ref, g_ref):
        g_ref[0] = lax.dot_general(
            q_ref[...], k_ref[...], (((1,), (1,)), ((), ())),
            preferred_element_type=jnp.float32) * INV_SQRT_D

    return pl.pallas_call(
        body,
        grid=(NSLAB,),
        in_specs=[
            pl.BlockSpec((N, D), lambda b: (0, 0)),
            pl.BlockSpec((D, D), lambda b: (b, 0)),
        ],
        out_specs=pl.BlockSpec((1, N, D), lambda b: (b, 0, 0)),
        out_shape=jax.ShapeDtypeStruct((NSLAB, N, D), jnp.float32),
    )(q, kp)


def _tc_combine(x, aggp, gatp, w0, w1):
    BR = 1000

    def body(x_ref, a_ref, g_ref, w0_ref, w1_ref, o_ref):
        xb = x_ref[...]
        agg = a_ref[0] + a_ref[1]
        gat = g_ref[0] + g_ref[1]
        gcn = (jnp.dot(xb, w0_ref[...], preferred_element_type=jnp.float32)
               + jnp.dot(agg, w1_ref[...],
                         preferred_element_type=jnp.float32))
        o_ref[...] = (gcn * (1.0 / N) + xb * ((N - 1.0) / N)
                      - gat * (1.0 / float(N) ** 3))

    return pl.pallas_call(
        body,
        grid=(N // BR,),
        in_specs=[
            pl.BlockSpec((BR, D), lambda i: (i, 0)),
            pl.BlockSpec((NC, BR, D), lambda i: (0, i, 0)),
            pl.BlockSpec((NC, BR, D), lambda i: (0, i, 0)),
            pl.BlockSpec((D, D), lambda i: (0, 0)),
            pl.BlockSpec((D, D), lambda i: (0, 0)),
        ],
        out_specs=pl.BlockSpec((BR, D), lambda i: (i, 0)),
        out_shape=jax.ShapeDtypeStruct((N, D), jnp.float32),
    )(x, aggp, gatp, w0, w1)


def kernel(input, edge_index, edge_index_2, W0, W1, Wq, Wk, Wv):
    x = input
    src = edge_index[0].astype(jnp.int32).reshape(NCHUNK, EB)
    dst = edge_index[1].astype(jnp.int32).reshape(NCHUNK, EB)
    s2 = edge_index_2[0].astype(jnp.int32).reshape(NCHUNK, EB)
    d2 = edge_index_2[1].astype(jnp.int32).reshape(NCHUNK, EB)
    zblk = jnp.zeros((BLKR, D), jnp.float32)
    q, k, v = _tc_qkv(x, Wq, Wk, Wv)
    kp = jnp.pad(k, ((0, KPAD - N), (0, 0))).astype(jnp.bfloat16)
    gsc = _tc_scores(q.astype(jnp.bfloat16), kp).reshape(NSLAB * N * D)
    aggp = _sc_agg(x, src, dst, zblk)
    zblk2 = zblk + aggp[0, :BLKR, :] * 0.0
    gatp = _sc_attn(gsc, v, s2, d2, zblk2)
    return _tc_combine(x, aggp, gatp, W0, W1)
